# baseline (speedup 1.0000x reference)
import functools

import jax
import jax.numpy as jnp
from jax import lax
from jax.experimental import pallas as pl
from jax.experimental.pallas import tpu as pltpu
from jax.experimental.pallas import tpu_sc as plsc

N = 10000
E = 320000
IN_DIM = 128
HID = 16

NC = 2
NS = 16
NW = NC * NS
CH = 128
TOTCH = E // CH
NCHUNK = 80
BIG = TOTCH - 78 * NW
NPAD = 10240
ROWS_PER_TILE = NPAD // NS
NBUF = 8
DELAY = 4

NPK = N * HID // 128
NPADPK = NPAD * HID // 128

_mesh = plsc.VectorSubcoreMesh(core_axis_name="c", subcore_axis_name="s")
_sc_params = pltpu.CompilerParams(use_tc_tiling_on_sc=False)


def _stage_indices(ei_hbm, row, idx_v, wid):
    start = 78 * wid + jnp.minimum(wid, BIG)
    big = wid < BIG

    @pl.when(big)
    def _():
        pltpu.sync_copy(ei_hbm.at[row].at[pl.ds(start, 79)],
                        idx_v.at[pl.ds(0, 79)])

    @pl.when(jnp.logical_not(big))
    def _():
        pltpu.sync_copy(ei_hbm.at[row].at[pl.ds(start, 78)],
                        idx_v.at[pl.ds(0, 78)])

    fill = jnp.zeros((16,), jnp.int32) if row == 0 else (
        N + lax.iota(jnp.int32, 16))
    for k in range(CH // 16):
        idx_v[NCHUNK - 1, 16 * k:16 * (k + 1)] = fill

        @pl.when(jnp.logical_not(big))
        def _():
            idx_v[NCHUNK - 2, 16 * k:16 * (k + 1)] = fill


def _sc_degree(ei_hbm, zeros_hbm, ones_hbm, out_hbm, dst_v, ones_v, acc_sh, sem):
    c = lax.axis_index("c")
    s = lax.axis_index("s")
    wid = s * NC + c
    base = s * ROWS_PER_TILE
    pltpu.sync_copy(zeros_hbm, acc_sh.at[pl.ds(base, ROWS_PER_TILE)])
    pltpu.sync_copy(ones_hbm, ones_v)
    _stage_indices(ei_hbm, 1, dst_v, wid)
    plsc.subcore_barrier()

    def fire(j, carry):
        pltpu.make_async_copy(ones_v, acc_sh.at[dst_v.at[j]], sem).start(add=True)
        return carry

    lax.fori_loop(0, NCHUNK, fire, 0)

    def drain(j, carry):
        pltpu.make_async_copy(ones_v, acc_sh.at[dst_v.at[j]], sem).wait()
        return carry

    lax.fori_loop(0, NCHUNK, drain, 0)
    plsc.subcore_barrier()
    pltpu.sync_copy(acc_sh.at[pl.ds(base, ROWS_PER_TILE)],
                    out_hbm.at[c].at[pl.ds(base, ROWS_PER_TILE)])


def _sc_scatter(g_hbm, ei_hbm, zeros_hbm, out_hbm,
                src_v, dst_v, rows_v, acc_sh, gsem, ssem):
    c = lax.axis_index("c")
    s = lax.axis_index("s")
    wid = s * NC + c
    base = s * ROWS_PER_TILE
    pltpu.sync_copy(zeros_hbm, acc_sh.at[pl.ds(base, ROWS_PER_TILE)])
    _stage_indices(ei_hbm, 0, src_v, wid)
    _stage_indices(ei_hbm, 1, dst_v, wid)
    plsc.subcore_barrier()

    def gather(j, b):
        pltpu.make_async_copy(
            g_hbm.at[src_v.at[j]], rows_v.at[b], gsem.at[b]).start()

    def gather_wait(j, b):
        pltpu.make_async_copy(
            g_hbm.at[src_v.at[j]], rows_v.at[b], gsem.at[b]).wait()

    def scatter(j, b):
        pltpu.make_async_copy(
            rows_v.at[b], acc_sh.at[dst_v.at[j]], ssem.at[b]).start(add=True)

    def scatter_wait(j, b):
        pltpu.make_async_copy(
            rows_v.at[b], acc_sh.at[dst_v.at[j]], ssem.at[b]).wait()

    for b in range(NBUF):
        gather(b, b)

    def body(o, carry):
        for b in range(NBUF):
            j = o * NBUF + b
            gather_wait(j, b)
            scatter(j, b)
            bd = (b - DELAY) % NBUF
            jd = j - DELAY

            @pl.when(jnp.logical_and(jd >= 0, jd + NBUF < NCHUNK))
            def _():
                scatter_wait(jd, bd)
                gather(jd + NBUF, bd)

        return carry

    lax.fori_loop(0, NCHUNK // NBUF, body, 0)
    for b in range(NBUF):
        scatter_wait(NCHUNK - NBUF + b, b)
    plsc.subcore_barrier()
    pltpu.sync_copy(acc_sh.at[pl.ds(base, ROWS_PER_TILE)],
                    out_hbm.at[c].at[pl.ds(base, ROWS_PER_TILE)])


def _tc1a(x_ref, w1e_ref, h8_ref):
    h8_ref[...] = jnp.dot(x_ref[...], w1e_ref[...],
                          preferred_element_type=jnp.float32)


def _tc1b(degp_ref, h8_ref, g1_ref, dinv_ref):
    degp = degp_ref[...]
    deg = degp[0, :NPK, :] + degp[1, :NPK, :] + 1.0
    dinv = lax.rsqrt(deg)
    h8r = h8_ref[...].reshape(NPK, 8, 128)
    lane = lax.broadcasted_iota(jnp.int32, (NPK, 128), 1) // 16
    hp = h8r[:, 0, :]
    for a in range(1, 8):
        hp = jnp.where(lane == a, h8r[:, a, :], hp)
    g1_ref[...] = hp * dinv
    dinv_ref[...] = dinv


def _tc2(accp_ref, g1_ref, dinv_ref, b1_ref, k2_ref, g2_ref):
    accp = accp_ref[...]
    acc = accp[0, :NPK, :] + accp[1, :NPK, :]
    dinv = dinv_ref[...]
    h1 = jnp.maximum(dinv * (acc + g1_ref[...]) + b1_ref[...], 0.0)
    g2_ref[...] = jnp.dot(h1, k2_ref[...],
                          preferred_element_type=jnp.float32) * dinv


def _tc3(accp_ref, g2_ref, dinv_ref, b2_ref, kfc_ref, bfc_ref, out_ref):
    accp = accp_ref[...]
    acc = accp[0, :NPK, :] + accp[1, :NPK, :]
    dinv = dinv_ref[...]
    h2 = jnp.maximum(dinv * (acc + g2_ref[...]) + b2_ref[...], 0.0)
    out_ref[...] = jnp.dot(h2, kfc_ref[...],
                           preferred_element_type=jnp.float32) + bfc_ref[...]


_deg_call = functools.partial(
    pl.kernel,
    mesh=_mesh,
    compiler_params=_sc_params,
    out_type=jax.ShapeDtypeStruct((NC, NPAD, HID), jnp.float32),
    scratch_types=[
        pltpu.VMEM((NCHUNK, CH), jnp.int32),
        pltpu.VMEM((CH, HID), jnp.float32),
        pltpu.VMEM_SHARED((NPAD, HID), jnp.float32),
        pltpu.SemaphoreType.DMA,
    ],
)(_sc_degree)

_scatter_call = functools.partial(
    pl.kernel,
    mesh=_mesh,
    compiler_params=_sc_params,
    out_type=jax.ShapeDtypeStruct((NC, NPAD, HID), jnp.float32),
    scratch_types=[
        pltpu.VMEM((NCHUNK, CH), jnp.int32),
        pltpu.VMEM((NCHUNK, CH), jnp.int32),
        pltpu.VMEM((NBUF, CH, HID), jnp.float32),
        pltpu.VMEM_SHARED((NPAD, HID), jnp.float32),
        pltpu.SemaphoreType.DMA((NBUF,)),
        pltpu.SemaphoreType.DMA((NBUF,)),
    ],
)(_sc_scatter)


def kernel(x, edge_index, W1, b1, W2, b2, Wfc, bfc):
    ei3 = edge_index.astype(jnp.int32).reshape(2, TOTCH, CH)
    zeros_tile = jnp.zeros((ROWS_PER_TILE, HID), jnp.float32)
    ones_chunk = jnp.ones((CH, HID), jnp.float32)
    eye8 = jnp.eye(8, dtype=jnp.float32)
    W1e = jnp.tile(W1, (1, 8))
    K2 = jnp.kron(eye8, W2)
    KFC = jnp.kron(eye8, Wfc)
    b1t = jnp.tile(b1, 8).reshape(1, 128)
    b2t = jnp.tile(b2, 8).reshape(1, 128)
    bfcr = bfc.reshape(1, 1)

    degp = _deg_call(ei3, zeros_tile, ones_chunk).reshape(NC, NPADPK, 128)

    h8 = pl.pallas_call(
        _tc1a,
        out_shape=jax.ShapeDtypeStruct((N, 128), jnp.float32),
    )(x, W1e)

    g1p, dinvp = pl.pallas_call(
        _tc1b,
        out_shape=(
            jax.ShapeDtypeStruct((NPK, 128), jnp.float32),
            jax.ShapeDtypeStruct((NPK, 128), jnp.float32),
        ),
    )(degp, h8)

    acc1p = _scatter_call(
        g1p.reshape(N, HID), ei3, zeros_tile).reshape(NC, NPADPK, 128)

    g2p = pl.pallas_call(
        _tc2,
        out_shape=jax.ShapeDtypeStruct((NPK, 128), jnp.float32),
    )(acc1p, g1p, dinvp, b1t, K2)

    acc2p = _scatter_call(
        g2p.reshape(N, HID), ei3, zeros_tile).reshape(NC, NPADPK, 128)

    outp = pl.pallas_call(
        _tc3,
        out_shape=jax.ShapeDtypeStruct((NPK, 8), jnp.float32),
    )(acc2p, g2p, dinvp, b2t, KFC, bfcr)

    return outp.reshape(N, 1)


# speedup vs baseline: 1.6136x; 1.6136x over previous
import functools

import jax
import jax.numpy as jnp
from jax import lax
from jax.experimental import pallas as pl
from jax.experimental.pallas import tpu as pltpu
from jax.experimental.pallas import tpu_sc as plsc

N = 10000
E = 320000
IN_DIM = 128
HID = 16

NC = 2
NS = 16
NW = NC * NS
CH = 128
TOTCH = E // CH
NCHUNK = 80
BIG = TOTCH - 78 * NW
NPAD = 10240
ROWS_PER_TILE = NPAD // NS
NBUF = 8
DELAY = 4

NPK = N * HID // 128
NPADPK = NPAD * HID // 128

_mesh = plsc.VectorSubcoreMesh(core_axis_name="c", subcore_axis_name="s")
_sc_params = pltpu.CompilerParams(use_tc_tiling_on_sc=False)


def _stage_indices(ei_hbm, row, idx_v, wid):
    pltpu.sync_copy(ei_hbm.at[row].at[wid], idx_v)


def _sc_degree(ei_hbm, zeros_hbm, ones_hbm, out_hbm, dst_v, ones_v, acc_sh, sem):
    c = lax.axis_index("c")
    s = lax.axis_index("s")
    wid = s * NC + c
    base = s * ROWS_PER_TILE
    pltpu.sync_copy(zeros_hbm, acc_sh.at[pl.ds(base, ROWS_PER_TILE)])
    pltpu.sync_copy(ones_hbm, ones_v)
    _stage_indices(ei_hbm, 1, dst_v, wid)
    plsc.subcore_barrier()

    def fire(j, carry):
        pltpu.make_async_copy(ones_v, acc_sh.at[dst_v.at[j]], sem).start(add=True)
        return carry

    lax.fori_loop(0, NCHUNK, fire, 0)

    def drain(j, carry):
        pltpu.make_async_copy(ones_v, acc_sh.at[dst_v.at[j]], sem).wait()
        return carry

    lax.fori_loop(0, NCHUNK, drain, 0)
    plsc.subcore_barrier()
    pltpu.sync_copy(acc_sh.at[pl.ds(base, ROWS_PER_TILE)],
                    out_hbm.at[c].at[pl.ds(base, ROWS_PER_TILE)])


def _sc_scatter(g_hbm, ei_hbm, zeros_hbm, out_hbm,
                src_v, dst_v, rows_v, acc_sh, gsem, ssem):
    c = lax.axis_index("c")
    s = lax.axis_index("s")
    wid = s * NC + c
    base = s * ROWS_PER_TILE
    pltpu.sync_copy(zeros_hbm, acc_sh.at[pl.ds(base, ROWS_PER_TILE)])
    _stage_indices(ei_hbm, 0, src_v, wid)
    _stage_indices(ei_hbm, 1, dst_v, wid)
    plsc.subcore_barrier()

    def gather(j, b):
        pltpu.make_async_copy(
            g_hbm.at[src_v.at[j]], rows_v.at[b], gsem.at[b]).start()

    def gather_wait(j, b):
        pltpu.make_async_copy(
            g_hbm.at[src_v.at[j]], rows_v.at[b], gsem.at[b]).wait()

    def scatter(j, b):
        pltpu.make_async_copy(
            rows_v.at[b], acc_sh.at[dst_v.at[j]], ssem.at[b]).start(add=True)

    def scatter_wait(j, b):
        pltpu.make_async_copy(
            rows_v.at[b], acc_sh.at[dst_v.at[j]], ssem.at[b]).wait()

    for b in range(NBUF):
        gather(b, b)

    def body(o, carry):
        for b in range(NBUF):
            j = o * NBUF + b
            gather_wait(j, b)
            scatter(j, b)
            bd = (b - DELAY) % NBUF
            jd = j - DELAY

            @pl.when(jnp.logical_and(jd >= 0, jd + NBUF < NCHUNK))
            def _():
                scatter_wait(jd, bd)
                gather(jd + NBUF, bd)

        return carry

    lax.fori_loop(0, NCHUNK // NBUF, body, 0)
    for b in range(NBUF):
        scatter_wait(NCHUNK - NBUF + b, b)
    plsc.subcore_barrier()
    pltpu.sync_copy(acc_sh.at[pl.ds(base, ROWS_PER_TILE)],
                    out_hbm.at[c].at[pl.ds(base, ROWS_PER_TILE)])


def _tc1a(x_ref, w1e_ref, h8_ref):
    h8_ref[...] = jnp.dot(x_ref[...], w1e_ref[...],
                          preferred_element_type=jnp.float32)


def _tc1b(degp_ref, h8_ref, g1_ref, dinv_ref):
    degp = degp_ref[...]
    deg = degp[0, :NPK, :] + degp[1, :NPK, :] + 1.0
    dinv = lax.rsqrt(deg)
    h8r = h8_ref[...].reshape(NPK, 8, 128)
    lane = lax.broadcasted_iota(jnp.int32, (NPK, 128), 1) // 16
    hp = h8r[:, 0, :]
    for a in range(1, 8):
        hp = jnp.where(lane == a, h8r[:, a, :], hp)
    g1_ref[...] = hp * dinv
    dinv_ref[...] = dinv


def _tc2(accp_ref, g1_ref, dinv_ref, b1_ref, k2_ref, g2_ref):
    accp = accp_ref[...]
    acc = accp[0, :NPK, :] + accp[1, :NPK, :]
    dinv = dinv_ref[...]
    h1 = jnp.maximum(dinv * (acc + g1_ref[...]) + b1_ref[...], 0.0)
    g2_ref[...] = jnp.dot(h1, k2_ref[...],
                          preferred_element_type=jnp.float32) * dinv


def _tc3(accp_ref, g2_ref, dinv_ref, b2_ref, kfc_ref, bfc_ref, out_ref):
    accp = accp_ref[...]
    acc = accp[0, :NPK, :] + accp[1, :NPK, :]
    dinv = dinv_ref[...]
    h2 = jnp.maximum(dinv * (acc + g2_ref[...]) + b2_ref[...], 0.0)
    out_ref[...] = jnp.dot(h2, kfc_ref[...],
                           preferred_element_type=jnp.float32) + bfc_ref[...]


_deg_call = functools.partial(
    pl.kernel,
    mesh=_mesh,
    compiler_params=_sc_params,
    out_type=jax.ShapeDtypeStruct((NC, NPAD, HID), jnp.float32),
    scratch_types=[
        pltpu.VMEM((NCHUNK, CH), jnp.int32),
        pltpu.VMEM((CH, HID), jnp.float32),
        pltpu.VMEM_SHARED((NPAD, HID), jnp.float32),
        pltpu.SemaphoreType.DMA,
    ],
)(_sc_degree)

_scatter_call = functools.partial(
    pl.kernel,
    mesh=_mesh,
    compiler_params=_sc_params,
    out_type=jax.ShapeDtypeStruct((NC, NPAD, HID), jnp.float32),
    scratch_types=[
        pltpu.VMEM((NCHUNK, CH), jnp.int32),
        pltpu.VMEM((NCHUNK, CH), jnp.int32),
        pltpu.VMEM((NBUF, CH, HID), jnp.float32),
        pltpu.VMEM_SHARED((NPAD, HID), jnp.float32),
        pltpu.SemaphoreType.DMA((NBUF,)),
        pltpu.SemaphoreType.DMA((NBUF,)),
    ],
)(_sc_scatter)


def kernel(x, edge_index, W1, b1, W2, b2, Wfc, bfc):
    ei = edge_index.astype(jnp.int32)
    pad = NW * NCHUNK * CH - E
    pad_src = (jnp.arange(pad, dtype=jnp.int32) * 37) % N
    pad_dst = N + (jnp.arange(pad, dtype=jnp.int32) % (NPAD - N))
    ei3 = jnp.concatenate(
        [ei, jnp.stack([pad_src, pad_dst])], axis=1).reshape(2, NW, NCHUNK, CH)
    zeros_tile = jnp.zeros((ROWS_PER_TILE, HID), jnp.float32)
    ones_chunk = jnp.ones((CH, HID), jnp.float32)
    eye8 = jnp.eye(8, dtype=jnp.float32)
    W1e = jnp.tile(W1, (1, 8))
    K2 = jnp.kron(eye8, W2)
    KFC = jnp.kron(eye8, Wfc)
    b1t = jnp.tile(b1, 8).reshape(1, 128)
    b2t = jnp.tile(b2, 8).reshape(1, 128)
    bfcr = bfc.reshape(1, 1)

    degp = _deg_call(ei3, zeros_tile, ones_chunk).reshape(NC, NPADPK, 128)

    h8 = pl.pallas_call(
        _tc1a,
        out_shape=jax.ShapeDtypeStruct((N, 128), jnp.float32),
    )(x, W1e)

    g1p, dinvp = pl.pallas_call(
        _tc1b,
        out_shape=(
            jax.ShapeDtypeStruct((NPK, 128), jnp.float32),
            jax.ShapeDtypeStruct((NPK, 128), jnp.float32),
        ),
    )(degp, h8)

    acc1p = _scatter_call(
        g1p.reshape(N, HID), ei3, zeros_tile).reshape(NC, NPADPK, 128)

    g2p = pl.pallas_call(
        _tc2,
        out_shape=jax.ShapeDtypeStruct((NPK, 128), jnp.float32),
    )(acc1p, g1p, dinvp, b1t, K2)

    acc2p = _scatter_call(
        g2p.reshape(N, HID), ei3, zeros_tile).reshape(NC, NPADPK, 128)

    outp = pl.pallas_call(
        _tc3,
        out_shape=jax.ShapeDtypeStruct((NPK, 8), jnp.float32),
    )(acc2p, g2p, dinvp, b2t, KFC, bfcr)

    return outp.reshape(N, 1)


# speedup vs baseline: 1.7665x; 1.0947x over previous
import functools

import jax
import jax.numpy as jnp
from jax import lax
from jax.experimental import pallas as pl
from jax.experimental.pallas import tpu as pltpu
from jax.experimental.pallas import tpu_sc as plsc

N = 10000
E = 320000
IN_DIM = 128
HID = 16

NC = 2
NS = 16
NW = NC * NS
CH = 128
TOTCH = E // CH
NCHUNK = 80
BIG = TOTCH - 78 * NW
NPAD = 10240
ROWS_PER_TILE = NPAD // NS
NBUF = 8
DELAY = 4

NPK = N * HID // 128
NPADPK = NPAD * HID // 128

_mesh = plsc.VectorSubcoreMesh(core_axis_name="c", subcore_axis_name="s")
_sc_params = pltpu.CompilerParams(use_tc_tiling_on_sc=False)


def _stage_indices(ei_hbm, row, idx_v, wid):
    pltpu.sync_copy(ei_hbm.at[row].at[wid], idx_v)


def _sc_degree(ei_hbm, zeros_hbm, ones_hbm, out_hbm, dst_v, ones_v, acc_sh, sem):
    c = lax.axis_index("c")
    s = lax.axis_index("s")
    wid = s * NC + c
    base = s * ROWS_PER_TILE
    pltpu.sync_copy(zeros_hbm, acc_sh.at[pl.ds(base, ROWS_PER_TILE)])
    pltpu.sync_copy(ones_hbm, ones_v)
    _stage_indices(ei_hbm, 1, dst_v, wid)
    plsc.subcore_barrier()

    def fire(j, carry):
        pltpu.make_async_copy(ones_v, acc_sh.at[dst_v.at[j]], sem).start(add=True)
        return carry

    lax.fori_loop(0, NCHUNK, fire, 0)

    def drain(j, carry):
        pltpu.make_async_copy(ones_v, acc_sh.at[dst_v.at[j]], sem).wait()
        return carry

    lax.fori_loop(0, NCHUNK, drain, 0)
    plsc.subcore_barrier()
    pltpu.sync_copy(acc_sh.at[pl.ds(base, ROWS_PER_TILE)],
                    out_hbm.at[c].at[pl.ds(base, ROWS_PER_TILE)])


def _sc_scatter(g_hbm, ei_hbm, zeros_hbm, out_hbm,
                src_v, dst_v, rows_v, acc_sh, g_sh, gsem, ssem):
    c = lax.axis_index("c")
    s = lax.axis_index("s")
    wid = s * NC + c
    base = s * ROWS_PER_TILE
    pltpu.sync_copy(zeros_hbm, acc_sh.at[pl.ds(base, ROWS_PER_TILE)])
    gbase = s * (N // NS)
    pltpu.sync_copy(g_hbm.at[pl.ds(gbase, N // NS)],
                    g_sh.at[pl.ds(gbase, N // NS)])
    _stage_indices(ei_hbm, 0, src_v, wid)
    _stage_indices(ei_hbm, 1, dst_v, wid)
    plsc.subcore_barrier()

    def gather(j, b):
        pltpu.make_async_copy(
            g_sh.at[src_v.at[j]], rows_v.at[b], gsem.at[b]).start()

    def gather_wait(j, b):
        pltpu.make_async_copy(
            g_sh.at[src_v.at[j]], rows_v.at[b], gsem.at[b]).wait()

    def scatter(j, b):
        pltpu.make_async_copy(
            rows_v.at[b], acc_sh.at[dst_v.at[j]], ssem.at[b]).start(add=True)

    def scatter_wait(j, b):
        pltpu.make_async_copy(
            rows_v.at[b], acc_sh.at[dst_v.at[j]], ssem.at[b]).wait()

    for b in range(NBUF):
        gather(b, b)

    def body(o, carry):
        for b in range(NBUF):
            j = o * NBUF + b
            gather_wait(j, b)
            scatter(j, b)
            bd = (b - DELAY) % NBUF
            jd = j - DELAY

            @pl.when(jnp.logical_and(jd >= 0, jd + NBUF < NCHUNK))
            def _():
                scatter_wait(jd, bd)
                gather(jd + NBUF, bd)

        return carry

    lax.fori_loop(0, NCHUNK // NBUF, body, 0)
    for b in range(NBUF):
        scatter_wait(NCHUNK - NBUF + b, b)
    plsc.subcore_barrier()
    pltpu.sync_copy(acc_sh.at[pl.ds(base, ROWS_PER_TILE)],
                    out_hbm.at[c].at[pl.ds(base, ROWS_PER_TILE)])


def _tc1a(x_ref, w1e_ref, h8_ref):
    h8_ref[...] = jnp.dot(x_ref[...], w1e_ref[...],
                          preferred_element_type=jnp.float32)


def _tc1b(degp_ref, h8_ref, g1_ref, dinv_ref):
    degp = degp_ref[...]
    deg = degp[0, :NPK, :] + degp[1, :NPK, :] + 1.0
    dinv = lax.rsqrt(deg)
    h8r = h8_ref[...].reshape(NPK, 8, 128)
    lane = lax.broadcasted_iota(jnp.int32, (NPK, 128), 1) // 16
    hp = h8r[:, 0, :]
    for a in range(1, 8):
        hp = jnp.where(lane == a, h8r[:, a, :], hp)
    g1_ref[...] = hp * dinv
    dinv_ref[...] = dinv


def _tc2(accp_ref, g1_ref, dinv_ref, b1_ref, k2_ref, g2_ref):
    accp = accp_ref[...]
    acc = accp[0, :NPK, :] + accp[1, :NPK, :]
    dinv = dinv_ref[...]
    h1 = jnp.maximum(dinv * (acc + g1_ref[...]) + b1_ref[...], 0.0)
    g2_ref[...] = jnp.dot(h1, k2_ref[...],
                          preferred_element_type=jnp.float32) * dinv


def _tc3(accp_ref, g2_ref, dinv_ref, b2_ref, kfc_ref, bfc_ref, out_ref):
    accp = accp_ref[...]
    acc = accp[0, :NPK, :] + accp[1, :NPK, :]
    dinv = dinv_ref[...]
    h2 = jnp.maximum(dinv * (acc + g2_ref[...]) + b2_ref[...], 0.0)
    out_ref[...] = jnp.dot(h2, kfc_ref[...],
                           preferred_element_type=jnp.float32) + bfc_ref[...]


_deg_call = functools.partial(
    pl.kernel,
    mesh=_mesh,
    compiler_params=_sc_params,
    out_type=jax.ShapeDtypeStruct((NC, NPAD, HID), jnp.float32),
    scratch_types=[
        pltpu.VMEM((NCHUNK, CH), jnp.int32),
        pltpu.VMEM((CH, HID), jnp.float32),
        pltpu.VMEM_SHARED((NPAD, HID), jnp.float32),
        pltpu.SemaphoreType.DMA,
    ],
)(_sc_degree)

_scatter_call = functools.partial(
    pl.kernel,
    mesh=_mesh,
    compiler_params=_sc_params,
    out_type=jax.ShapeDtypeStruct((NC, NPAD, HID), jnp.float32),
    scratch_types=[
        pltpu.VMEM((NCHUNK, CH), jnp.int32),
        pltpu.VMEM((NCHUNK, CH), jnp.int32),
        pltpu.VMEM((NBUF, CH, HID), jnp.float32),
        pltpu.VMEM_SHARED((NPAD, HID), jnp.float32),
        pltpu.VMEM_SHARED((N, HID), jnp.float32),
        pltpu.SemaphoreType.DMA((NBUF,)),
        pltpu.SemaphoreType.DMA((NBUF,)),
    ],
)(_sc_scatter)


def kernel(x, edge_index, W1, b1, W2, b2, Wfc, bfc):
    ei = edge_index.astype(jnp.int32)
    pad = NW * NCHUNK * CH - E
    pad_src = (jnp.arange(pad, dtype=jnp.int32) * 37) % N
    pad_dst = N + (jnp.arange(pad, dtype=jnp.int32) % (NPAD - N))
    ei3 = jnp.concatenate(
        [ei, jnp.stack([pad_src, pad_dst])], axis=1).reshape(2, NW, NCHUNK, CH)
    zeros_tile = jnp.zeros((ROWS_PER_TILE, HID), jnp.float32)
    ones_chunk = jnp.ones((CH, HID), jnp.float32)
    eye8 = jnp.eye(8, dtype=jnp.float32)
    W1e = jnp.tile(W1, (1, 8))
    K2 = jnp.kron(eye8, W2)
    KFC = jnp.kron(eye8, Wfc)
    b1t = jnp.tile(b1, 8).reshape(1, 128)
    b2t = jnp.tile(b2, 8).reshape(1, 128)
    bfcr = bfc.reshape(1, 1)

    degp = _deg_call(ei3, zeros_tile, ones_chunk).reshape(NC, NPADPK, 128)

    h8 = pl.pallas_call(
        _tc1a,
        out_shape=jax.ShapeDtypeStruct((N, 128), jnp.float32),
    )(x, W1e)

    g1p, dinvp = pl.pallas_call(
        _tc1b,
        out_shape=(
            jax.ShapeDtypeStruct((NPK, 128), jnp.float32),
            jax.ShapeDtypeStruct((NPK, 128), jnp.float32),
        ),
    )(degp, h8)

    acc1p = _scatter_call(
        g1p.reshape(N, HID), ei3, zeros_tile).reshape(NC, NPADPK, 128)

    g2p = pl.pallas_call(
        _tc2,
        out_shape=jax.ShapeDtypeStruct((NPK, 128), jnp.float32),
    )(acc1p, g1p, dinvp, b1t, K2)

    acc2p = _scatter_call(
        g2p.reshape(N, HID), ei3, zeros_tile).reshape(NC, NPADPK, 128)

    outp = pl.pallas_call(
        _tc3,
        out_shape=jax.ShapeDtypeStruct((NPK, 8), jnp.float32),
    )(acc2p, g2p, dinvp, b2t, KFC, bfcr)

    return outp.reshape(N, 1)


# speedup vs baseline: 1.7992x; 1.0185x over previous
import functools

import jax
import jax.numpy as jnp
from jax import lax
from jax.experimental import pallas as pl
from jax.experimental.pallas import tpu as pltpu
from jax.experimental.pallas import tpu_sc as plsc

N = 10000
E = 320000
IN_DIM = 128
HID = 16

NC = 2
NS = 16
NW = NC * NS
CH = 128
TOTCH = E // CH
NCHUNK = 80
BIG = TOTCH - 78 * NW
NPAD = 10240
ROWS_PER_TILE = NPAD // NS
NBUF = 8
DELAY = 4

NPK = N * HID // 128
NPADPK = NPAD * HID // 128

_mesh = plsc.VectorSubcoreMesh(core_axis_name="c", subcore_axis_name="s")
_sc_params = pltpu.CompilerParams(use_tc_tiling_on_sc=False)


def _stage_indices(ei_hbm, row, idx_v, wid):
    pltpu.sync_copy(ei_hbm.at[row].at[wid], idx_v)


def _sc_degree(ei_hbm, zeros_hbm, ones_hbm, out_hbm, dst_v, ones_v, acc_sh, sem):
    c = lax.axis_index("c")
    s = lax.axis_index("s")
    wid = s * NC + c
    base = s * ROWS_PER_TILE
    pltpu.sync_copy(zeros_hbm, acc_sh.at[pl.ds(base, ROWS_PER_TILE)])
    pltpu.sync_copy(ones_hbm, ones_v)
    _stage_indices(ei_hbm, 1, dst_v, wid)
    plsc.subcore_barrier()

    def fire(j, carry):
        pltpu.make_async_copy(ones_v, acc_sh.at[dst_v.at[j]], sem).start(add=True)
        return carry

    lax.fori_loop(0, NCHUNK, fire, 0)

    def drain(j, carry):
        pltpu.make_async_copy(ones_v, acc_sh.at[dst_v.at[j]], sem).wait()
        return carry

    lax.fori_loop(0, NCHUNK, drain, 0)
    plsc.subcore_barrier()
    pltpu.sync_copy(acc_sh.at[pl.ds(base, ROWS_PER_TILE)],
                    out_hbm.at[c].at[pl.ds(base, ROWS_PER_TILE)])


def _sc_scatter(g_hbm, ei_hbm, zeros_hbm, out_hbm,
                src_v, dst_v, rows_v, acc_sh, g_sh, gsem, ssem):
    c = lax.axis_index("c")
    s = lax.axis_index("s")
    wid = s * NC + c
    base = s * ROWS_PER_TILE
    pltpu.sync_copy(zeros_hbm, acc_sh.at[pl.ds(base, ROWS_PER_TILE)])
    gbase = s * (N // NS)
    pltpu.sync_copy(g_hbm.at[pl.ds(gbase, N // NS)],
                    g_sh.at[pl.ds(gbase, N // NS)])
    _stage_indices(ei_hbm, 0, src_v, wid)
    _stage_indices(ei_hbm, 1, dst_v, wid)
    plsc.subcore_barrier()

    def gather(j, b):
        pltpu.make_async_copy(
            g_sh.at[src_v.at[j]], rows_v.at[b], gsem.at[b]).start()

    def gather_wait(j, b):
        pltpu.make_async_copy(
            g_sh.at[src_v.at[j]], rows_v.at[b], gsem.at[b]).wait()

    def scatter(j, b):
        pltpu.make_async_copy(
            rows_v.at[b], acc_sh.at[dst_v.at[j]], ssem.at[b]).start(add=True)

    def scatter_wait(j, b):
        pltpu.make_async_copy(
            rows_v.at[b], acc_sh.at[dst_v.at[j]], ssem.at[b]).wait()

    for b in range(NBUF):
        gather(b, b)

    def body(o, carry):
        for b in range(NBUF):
            j = o * NBUF + b
            gather_wait(j, b)
            scatter(j, b)
            bd = (b - DELAY) % NBUF
            jd = j - DELAY

            @pl.when(jnp.logical_and(jd >= 0, jd + NBUF < NCHUNK))
            def _():
                scatter_wait(jd, bd)
                gather(jd + NBUF, bd)

        return carry

    lax.fori_loop(0, NCHUNK // NBUF, body, 0)
    for b in range(NBUF):
        scatter_wait(NCHUNK - NBUF + b, b)
    plsc.subcore_barrier()
    pltpu.sync_copy(acc_sh.at[pl.ds(base, ROWS_PER_TILE)],
                    out_hbm.at[c].at[pl.ds(base, ROWS_PER_TILE)])


def _tc1a(x_ref, w1e_ref, h8_ref):
    h8_ref[...] = jnp.dot(x_ref[...], w1e_ref[...],
                          preferred_element_type=jnp.float32)


def _tc1b(degp_ref, h8_ref, g1_ref, dinv_ref):
    degp = degp_ref[...]
    deg8 = degp[0] + degp[1] + 1.0
    dinv8 = lax.rsqrt(deg8)
    lane8 = lax.broadcasted_iota(jnp.int32, (NPK, 128), 1) // 16
    dinv = jnp.broadcast_to(dinv8[:, 0:1], (NPK, 128))
    for a in range(1, 8):
        dinv = jnp.where(lane8 == a,
                         jnp.broadcast_to(dinv8[:, a:a + 1], (NPK, 128)), dinv)
    h8r = h8_ref[...].reshape(NPK, 8, 128)
    lane = lax.broadcasted_iota(jnp.int32, (NPK, 128), 1) // 16
    hp = h8r[:, 0, :]
    for a in range(1, 8):
        hp = jnp.where(lane == a, h8r[:, a, :], hp)
    g1_ref[...] = hp * dinv
    dinv_ref[...] = dinv


def _tc2(accp_ref, g1_ref, dinv_ref, b1_ref, k2_ref, g2_ref):
    accp = accp_ref[...]
    acc = accp[0, :NPK, :] + accp[1, :NPK, :]
    dinv = dinv_ref[...]
    h1 = jnp.maximum(dinv * (acc + g1_ref[...]) + b1_ref[...], 0.0)
    g2_ref[...] = jnp.dot(h1, k2_ref[...],
                          preferred_element_type=jnp.float32) * dinv


def _tc3(accp_ref, g2_ref, dinv_ref, b2_ref, kfc_ref, bfc_ref, out_ref):
    accp = accp_ref[...]
    acc = accp[0, :NPK, :] + accp[1, :NPK, :]
    dinv = dinv_ref[...]
    h2 = jnp.maximum(dinv * (acc + g2_ref[...]) + b2_ref[...], 0.0)
    out_ref[...] = jnp.dot(h2, kfc_ref[...],
                           preferred_element_type=jnp.float32) + bfc_ref[...]


_deg_call = functools.partial(
    pl.kernel,
    mesh=_mesh,
    compiler_params=_sc_params,
    out_type=jax.ShapeDtypeStruct((NC, NPAD), jnp.float32),
    scratch_types=[
        pltpu.VMEM((NCHUNK, CH), jnp.int32),
        pltpu.VMEM((CH,), jnp.float32),
        pltpu.VMEM_SHARED((NPAD,), jnp.float32),
        pltpu.SemaphoreType.DMA,
    ],
)(_sc_degree)

_scatter_call = functools.partial(
    pl.kernel,
    mesh=_mesh,
    compiler_params=_sc_params,
    out_type=jax.ShapeDtypeStruct((NC, NPAD, HID), jnp.float32),
    scratch_types=[
        pltpu.VMEM((NCHUNK, CH), jnp.int32),
        pltpu.VMEM((NCHUNK, CH), jnp.int32),
        pltpu.VMEM((NBUF, CH, HID), jnp.float32),
        pltpu.VMEM_SHARED((NPAD, HID), jnp.float32),
        pltpu.VMEM_SHARED((N, HID), jnp.float32),
        pltpu.SemaphoreType.DMA((NBUF,)),
        pltpu.SemaphoreType.DMA((NBUF,)),
    ],
)(_sc_scatter)


def kernel(x, edge_index, W1, b1, W2, b2, Wfc, bfc):
    ei = edge_index.astype(jnp.int32)
    pad = NW * NCHUNK * CH - E
    pad_src = (jnp.arange(pad, dtype=jnp.int32) * 37) % N
    pad_dst = N + (jnp.arange(pad, dtype=jnp.int32) % (NPAD - N))
    ei3 = jnp.concatenate(
        [ei, jnp.stack([pad_src, pad_dst])], axis=1).reshape(2, NW, NCHUNK, CH)
    zeros_tile = jnp.zeros((ROWS_PER_TILE, HID), jnp.float32)
    ones_chunk = jnp.ones((CH, HID), jnp.float32)
    eye8 = jnp.eye(8, dtype=jnp.float32)
    W1e = jnp.tile(W1, (1, 8))
    K2 = jnp.kron(eye8, W2)
    KFC = jnp.kron(eye8, Wfc)
    b1t = jnp.tile(b1, 8).reshape(1, 128)
    b2t = jnp.tile(b2, 8).reshape(1, 128)
    bfcr = bfc.reshape(1, 1)

    zeros1 = jnp.zeros((ROWS_PER_TILE,), jnp.float32)
    ones1 = jnp.ones((CH,), jnp.float32)
    degp = _deg_call(ei3, zeros1, ones1)[:, :N].reshape(NC, NPK, 8)

    h8 = pl.pallas_call(
        _tc1a,
        out_shape=jax.ShapeDtypeStruct((N, 128), jnp.float32),
    )(x, W1e)

    g1p, dinvp = pl.pallas_call(
        _tc1b,
        out_shape=(
            jax.ShapeDtypeStruct((NPK, 128), jnp.float32),
            jax.ShapeDtypeStruct((NPK, 128), jnp.float32),
        ),
    )(degp, h8)

    acc1p = _scatter_call(
        g1p.reshape(N, HID), ei3, zeros_tile).reshape(NC, NPADPK, 128)

    g2p = pl.pallas_call(
        _tc2,
        out_shape=jax.ShapeDtypeStruct((NPK, 128), jnp.float32),
    )(acc1p, g1p, dinvp, b1t, K2)

    acc2p = _scatter_call(
        g2p.reshape(N, HID), ei3, zeros_tile).reshape(NC, NPADPK, 128)

    outp = pl.pallas_call(
        _tc3,
        out_shape=jax.ShapeDtypeStruct((NPK, 8), jnp.float32),
    )(acc2p, g2p, dinvp, b2t, KFC, bfcr)

    return outp.reshape(N, 1)


# speedup vs baseline: 2.1982x; 1.2218x over previous
import functools

import jax
import jax.numpy as jnp
from jax import lax
from jax.experimental import pallas as pl
from jax.experimental.pallas import tpu as pltpu
from jax.experimental.pallas import tpu_sc as plsc

N = 10000
E = 320000
IN_DIM = 128
HID = 16

NC = 2
NS = 16
NW = NC * NS
CH = 128
TOTCH = E // CH
NCHUNK = 80
BIG = TOTCH - 78 * NW
NPAD = 10240
ROWS_PER_TILE = NPAD // NS
NBUF = 8
DELAY = 6

NPK = N * HID // 128
NPADPK = NPAD * HID // 128

_mesh = plsc.VectorSubcoreMesh(core_axis_name="c", subcore_axis_name="s")
_sc_params = pltpu.CompilerParams(use_tc_tiling_on_sc=False)


def _stage_indices(ei_hbm, row, idx_v, wid):
    pltpu.sync_copy(ei_hbm.at[row].at[wid], idx_v)


def _sc_degree(ei_hbm, zeros_hbm, ones_hbm, out_hbm, dst_v, ones_v, acc_sh, sem):
    c = lax.axis_index("c")
    s = lax.axis_index("s")
    wid = s * NC + c
    base = s * ROWS_PER_TILE
    pltpu.sync_copy(zeros_hbm, acc_sh.at[pl.ds(base, ROWS_PER_TILE)])
    pltpu.sync_copy(ones_hbm, ones_v)
    _stage_indices(ei_hbm, 1, dst_v, wid)
    plsc.subcore_barrier()

    def fire(j, carry):
        pltpu.make_async_copy(ones_v, acc_sh.at[dst_v.at[j]], sem).start(add=True)
        return carry

    lax.fori_loop(0, NCHUNK, fire, 0)

    def drain(j, carry):
        pltpu.make_async_copy(ones_v, acc_sh.at[dst_v.at[j]], sem).wait()
        return carry

    lax.fori_loop(0, NCHUNK, drain, 0)
    plsc.subcore_barrier()
    pltpu.sync_copy(acc_sh.at[pl.ds(base, ROWS_PER_TILE)],
                    out_hbm.at[c].at[pl.ds(base, ROWS_PER_TILE)])


def _sc_scatter(g_hbm, ei_hbm, zeros_hbm, out_hbm,
                src_v, dst_v, rows_v, acc_sh, g_sh, gsem, ssem):
    c = lax.axis_index("c")
    s = lax.axis_index("s")
    wid = s * NC + c
    base = s * ROWS_PER_TILE
    pltpu.sync_copy(zeros_hbm, acc_sh.at[pl.ds(base, ROWS_PER_TILE)])
    gbase = s * (N // NS)
    pltpu.sync_copy(g_hbm.at[pl.ds(gbase, N // NS)],
                    g_sh.at[pl.ds(gbase, N // NS)])
    _stage_indices(ei_hbm, 0, src_v, wid)
    _stage_indices(ei_hbm, 1, dst_v, wid)
    plsc.subcore_barrier()

    def gather(j, b):
        pltpu.make_async_copy(
            g_sh.at[src_v.at[j]], rows_v.at[b], gsem.at[b]).start()

    def gather_wait(j, b):
        pltpu.make_async_copy(
            g_sh.at[src_v.at[j]], rows_v.at[b], gsem.at[b]).wait()

    def scatter(j, b):
        pltpu.make_async_copy(
            rows_v.at[b], acc_sh.at[dst_v.at[j]], ssem.at[b]).start(add=True)

    def scatter_wait(j, b):
        pltpu.make_async_copy(
            rows_v.at[b], acc_sh.at[dst_v.at[j]], ssem.at[b]).wait()

    for b in range(NBUF):
        gather(b, b)

    def body(o, carry):
        for b in range(NBUF):
            j = o * NBUF + b
            gather_wait(j, b)
            scatter(j, b)
            bd = (b - DELAY) % NBUF
            jd = j - DELAY

            @pl.when(jnp.logical_and(jd >= 0, jd + NBUF < NCHUNK))
            def _():
                scatter_wait(jd, bd)
                gather(jd + NBUF, bd)

        return carry

    lax.fori_loop(0, NCHUNK // NBUF, body, 0)
    for b in range(NBUF):
        scatter_wait(NCHUNK - NBUF + b, b)
    plsc.subcore_barrier()
    pltpu.sync_copy(acc_sh.at[pl.ds(base, ROWS_PER_TILE)],
                    out_hbm.at[c].at[pl.ds(base, ROWS_PER_TILE)])


def _tc1a(x_ref, w1e_ref, h8_ref):
    h8_ref[...] = jnp.dot(x_ref[...], w1e_ref[...],
                          preferred_element_type=jnp.float32)


def _tc1b(degp_ref, h8_ref, g1_ref, dinv_ref):
    degp = degp_ref[...]
    deg8 = degp[0] + degp[1] + 1.0
    dinv8 = lax.rsqrt(deg8)
    lane8 = lax.broadcasted_iota(jnp.int32, (NPK, 128), 1) // 16
    dinv = jnp.broadcast_to(dinv8[:, 0:1], (NPK, 128))
    for a in range(1, 8):
        dinv = jnp.where(lane8 == a,
                         jnp.broadcast_to(dinv8[:, a:a + 1], (NPK, 128)), dinv)
    h8r = h8_ref[...].reshape(NPK, 8, 128)
    lane = lax.broadcasted_iota(jnp.int32, (NPK, 128), 1) // 16
    hp = h8r[:, 0, :]
    for a in range(1, 8):
        hp = jnp.where(lane == a, h8r[:, a, :], hp)
    g1_ref[...] = hp * dinv
    dinv_ref[...] = dinv


def _tc2(accp_ref, g1_ref, dinv_ref, b1_ref, k2_ref, g2_ref):
    accp = accp_ref[...]
    acc = accp[0, :NPK, :] + accp[1, :NPK, :]
    dinv = dinv_ref[...]
    h1 = jnp.maximum(dinv * (acc + g1_ref[...]) + b1_ref[...], 0.0)
    g2_ref[...] = jnp.dot(h1, k2_ref[...],
                          preferred_element_type=jnp.float32) * dinv


def _tc3(accp_ref, g2_ref, dinv_ref, b2_ref, kfc_ref, bfc_ref, out_ref):
    accp = accp_ref[...]
    acc = accp[0, :NPK, :] + accp[1, :NPK, :]
    dinv = dinv_ref[...]
    h2 = jnp.maximum(dinv * (acc + g2_ref[...]) + b2_ref[...], 0.0)
    out_ref[...] = jnp.dot(h2, kfc_ref[...],
                           preferred_element_type=jnp.float32) + bfc_ref[...]


_deg_call = functools.partial(
    pl.kernel,
    mesh=_mesh,
    compiler_params=_sc_params,
    out_type=jax.ShapeDtypeStruct((NC, NPAD), jnp.float32),
    scratch_types=[
        pltpu.VMEM((NCHUNK, CH), jnp.int32),
        pltpu.VMEM((CH,), jnp.float32),
        pltpu.VMEM_SHARED((NPAD,), jnp.float32),
        pltpu.SemaphoreType.DMA,
    ],
)(_sc_degree)

_scatter_call = functools.partial(
    pl.kernel,
    mesh=_mesh,
    compiler_params=_sc_params,
    out_type=jax.ShapeDtypeStruct((NC, NPAD, HID), jnp.float32),
    scratch_types=[
        pltpu.VMEM((NCHUNK, CH), jnp.int32),
        pltpu.VMEM((NCHUNK, CH), jnp.int32),
        pltpu.VMEM((NBUF, CH, HID), jnp.float32),
        pltpu.VMEM_SHARED((NPAD, HID), jnp.float32),
        pltpu.VMEM_SHARED((N, HID), jnp.float32),
        pltpu.SemaphoreType.DMA((NBUF,)),
        pltpu.SemaphoreType.DMA((NBUF,)),
    ],
)(_sc_scatter)


def kernel(x, edge_index, W1, b1, W2, b2, Wfc, bfc):
    ei = edge_index.astype(jnp.int32)
    pad = NW * NCHUNK * CH - E
    pad_src = (jnp.arange(pad, dtype=jnp.int32) * 37) % N
    pad_dst = N + (jnp.arange(pad, dtype=jnp.int32) % (NPAD - N))
    ei3 = jnp.concatenate(
        [ei, jnp.stack([pad_src, pad_dst])], axis=1).reshape(2, NW, NCHUNK, CH)
    zeros_tile = jnp.zeros((ROWS_PER_TILE, HID), jnp.float32)
    ones_chunk = jnp.ones((CH, HID), jnp.float32)
    eye8 = jnp.eye(8, dtype=jnp.float32)
    W1e = jnp.tile(W1, (1, 8))
    K2 = jnp.kron(eye8, W2)
    KFC = jnp.kron(eye8, Wfc)
    b1t = jnp.tile(b1, 8).reshape(1, 128)
    b2t = jnp.tile(b2, 8).reshape(1, 128)
    bfcr = bfc.reshape(1, 1)

    zeros1 = jnp.zeros((ROWS_PER_TILE,), jnp.float32)
    ones1 = jnp.ones((CH,), jnp.float32)
    degp = _deg_call(ei3, zeros1, ones1)[:, :N].reshape(NC, NPK, 8)

    h8 = pl.pallas_call(
        _tc1a,
        out_shape=jax.ShapeDtypeStruct((N, 128), jnp.float32),
    )(x, W1e)

    g1p, dinvp = pl.pallas_call(
        _tc1b,
        out_shape=(
            jax.ShapeDtypeStruct((NPK, 128), jnp.float32),
            jax.ShapeDtypeStruct((NPK, 128), jnp.float32),
        ),
    )(degp, h8)

    acc1p = _scatter_call(
        g1p.reshape(N, HID), ei3, zeros_tile).reshape(NC, NPADPK, 128)

    g2p = pl.pallas_call(
        _tc2,
        out_shape=jax.ShapeDtypeStruct((NPK, 128), jnp.float32),
    )(acc1p, g1p, dinvp, b1t, K2)

    acc2p = _scatter_call(
        g2p.reshape(N, HID), ei3, zeros_tile).reshape(NC, NPADPK, 128)

    outp = pl.pallas_call(
        _tc3,
        out_shape=jax.ShapeDtypeStruct((NPK, 8), jnp.float32),
    )(acc2p, g2p, dinvp, b2t, KFC, bfcr)

    return outp.reshape(N, 1)
